# baseline (speedup 1.0000x reference)
import functools

import jax
import jax.numpy as jnp
from jax import lax
from jax.experimental import pallas as pl
from jax.experimental.pallas import tpu as pltpu
from jax.experimental.pallas import tpu_sc as plsc

N_IMG, C, H, W = 6, 128, 64, 176
OCC = (200, 200, 16)
NVOX = OCC[0] * OCC[1] * OCC[2]
HW = H * W
HWP = HW + 8
SENT = HW

G = 128
NT = NVOX // G
NW = 32
KPW = 160
NBUF = 4
AHEAD = NBUF - 1
K_ITERS = (KPW + NBUF) // NBUF + 1
NCH = G // 16


def _table_body(in_ref, out_ref):
    out_ref[0, :HW] = in_ref[0].T
    out_ref[0, HW:] = jnp.zeros((HWP - HW, C), jnp.float32)


def _build_table(img3):
    return pl.pallas_call(
        _table_body,
        grid=(N_IMG,),
        in_specs=[pl.BlockSpec((1, C, HW), lambda n: (n, 0, 0))],
        out_specs=pl.BlockSpec((1, HWP, C), lambda n: (n, 0, 0)),
        out_shape=jax.ShapeDtypeStruct((N_IMG, HWP, C), jnp.float32),
    )(img3)


def _round_clip(v, hi):
    t = v.astype(jnp.int32)
    fr = v - t.astype(jnp.float32)
    one = jnp.full((16,), 1, jnp.int32)
    zero = jnp.zeros((16,), jnp.int32)
    up = jnp.where(fr > 0.5, one, jnp.where(fr == 0.5, t & one, zero))
    r = t + up
    return jnp.minimum(jnp.maximum(r, zero), jnp.full((16,), hi, jnp.int32))


def _sc_fused(pts_t, val32, table):
    mesh = plsc.VectorSubcoreMesh(core_axis_name="c", subcore_axis_name="s")

    @functools.partial(
        pl.kernel,
        mesh=mesh,
        out_type=jax.ShapeDtypeStruct((NVOX, C), jnp.float32),
        scratch_types=[
            pltpu.VMEM((NBUF, N_IMG, G), jnp.float32),
            pltpu.VMEM((NBUF, N_IMG, G), jnp.float32),
            pltpu.VMEM((NBUF, N_IMG, G), jnp.int32),
            pltpu.VMEM((NBUF, G), jnp.int32),
            pltpu.VMEM((NBUF, G, C), jnp.float32),
            pltpu.SemaphoreType.DMA((NBUF,)),
            pltpu.SemaphoreType.DMA((NBUF,)),
            pltpu.SemaphoreType.DMA((NBUF,)),
        ],
        compiler_params=pltpu.CompilerParams(use_tc_tiling_on_sc=True),
    )
    def k(pts_hbm, val_hbm, table_hbm, out_hbm,
          px_v, py_v, val_v, idx_v, rows_v, psem, gsem, ssem):
        wid = lax.axis_index("s") * 2 + lax.axis_index("c")
        start = wid * KPW
        vs = jnp.minimum(NT - start, KPW)

        def pstart(b, slot):
            v0 = (start + slot) * G
            for i in range(N_IMG):
                pltpu.async_copy(pts_hbm.at[i, 0, pl.ds(v0, G)],
                                 px_v.at[b, i], psem.at[b])
                pltpu.async_copy(pts_hbm.at[i, 1, pl.ds(v0, G)],
                                 py_v.at[b, i], psem.at[b])
                pltpu.async_copy(val_hbm.at[i, pl.ds(v0, G)],
                                 val_v.at[b, i], psem.at[b])

        def pwait(b):
            for i in range(N_IMG):
                pltpu.make_async_copy(pts_hbm.at[i, 0, pl.ds(0, G)],
                                      px_v.at[b, i], psem.at[b]).wait()
                pltpu.make_async_copy(pts_hbm.at[i, 1, pl.ds(0, G)],
                                      py_v.at[b, i], psem.at[b]).wait()
                pltpu.make_async_copy(val_hbm.at[i, pl.ds(0, G)],
                                      val_v.at[b, i], psem.at[b]).wait()

        def compute_idx(b):
            def chunk(j, carry):
                sl = pl.ds(j * 16, 16)
                gidx = jnp.full((16,), SENT, jnp.int32)
                for i in range(N_IMG):
                    x = _round_clip(px_v[b, i, sl], W - 1)
                    y = _round_clip(py_v[b, i, sl], H - 1)
                    pos = y * W + x + i * HWP
                    gidx = jnp.where(val_v[b, i, sl] != 0, pos, gidx)
                idx_v[b, sl] = gidx
                return carry

            lax.fori_loop(0, NCH, chunk, 0)

        def gstart(b, slot):
            pltpu.async_copy(table_hbm.at[idx_v.at[b]], rows_v.at[b],
                             gsem.at[b])

        def gwait(b):
            pltpu.make_async_copy(table_hbm.at[idx_v.at[b]], rows_v.at[b],
                                  gsem.at[b]).wait()

        def sstart(b, t):
            pltpu.async_copy(rows_v.at[b], out_hbm.at[pl.ds(t * G, G)],
                             ssem.at[b])

        def swait(b):
            pltpu.make_async_copy(rows_v.at[b], out_hbm.at[pl.ds(0, G)],
                                  ssem.at[b]).wait()

        for d in range(NBUF):

            @pl.when(d < vs)
            def _(d=d):
                pstart(d, d)

        for d in range(AHEAD):

            @pl.when(d < vs)
            def _(d=d):
                pwait(d)
                compute_idx(d)
                gstart(d, d)

        def body(k0, carry):
            for db in range(NBUF):
                slot = k0 * NBUF + db

                @pl.when(slot < vs)
                def _(slot=slot, db=db):
                    gwait(db)
                    sstart(db, start + slot)

                @pl.when((slot >= 1) & (slot - 1 < vs))
                def _(db=db):
                    swait((db + AHEAD) % NBUF)

                slot_a = slot + AHEAD
                ba = (db + AHEAD) % NBUF

                @pl.when(slot_a < vs)
                def _(slot_a=slot_a, ba=ba):
                    pwait(ba)
                    compute_idx(ba)
                    gstart(ba, slot_a)

                slot_p = slot + NBUF

                @pl.when(slot_p < vs)
                def _(slot_p=slot_p, db=db):
                    pstart(db, slot_p)

            return carry

        lax.fori_loop(0, K_ITERS, body, 0)

    return k(pts_t, val32, table)


def kernel(img_feats, points, valid):
    table = _build_table(img_feats.reshape(N_IMG, C, HW))
    table = table.reshape(N_IMG * HWP, C)

    pts_t = jnp.transpose(points, (0, 2, 1))
    val32 = valid.astype(jnp.int32)
    packed = _sc_fused(pts_t, val32, table)

    vol = packed.reshape(OCC[0], OCC[1], OCC[2], C)
    return jnp.transpose(vol, (3, 0, 1, 2))
